# baseline (speedup 1.0000x reference)
import jax
import jax.numpy as jnp
from jax import lax
from jax.experimental import pallas as pl
from jax.experimental.pallas import tpu as pltpu
from jax.experimental.pallas import tpu_sc as plsc

B = 16384
D = 64
V = 1000000
NW = 32
BPW = B // NW
BCOLS = 244
NR = 10
C = 1024
SB = 32
SBC = 64
PC = 32
JOFF = B + NW + 32
UROWS = B + NW
VROWS = JOFF + B + NW

_IOTA = lambda: lax.iota(jnp.int32, 16)


def _bucket(idx_hbm, idxbuf, wl_v, wl_b, done, lo, hi, sem):
    def block(blk, carry):
        pltpu.async_copy(
            idx_hbm.at[pl.ds(blk * 1024, 1024)], idxbuf, sem).wait()

        def chunk(k, carry):
            off, seen = carry
            iv = idxbuf[pl.ds(k * 16, 16)]
            bv = blk * 1024 + k * 16 + _IOTA()
            m = (iv >= lo) & (iv < hi)
            mcnt = plsc.all_reduce_population_count(m)[0]
            fast = (done == 0) & (off + 16 <= C)

            def fastf(_):
                @pl.when(mcnt > 0)
                def _():
                    plsc.store_compressed(
                        wl_v.at[pl.ds(off, 16)], iv, mask=m)
                    plsc.store_compressed(
                        wl_b.at[pl.ds(off, 16)], bv, mask=m)

                return mcnt

            def slowf(_):
                rank = seen + plsc.cumsum(jnp.where(m, 1, 0))
                keep = m & (rank > done) & (rank <= done + C)
                kcnt = plsc.all_reduce_population_count(keep)[0]

                @pl.when(kcnt > 0)
                def _():
                    plsc.store_compressed(
                        wl_v.at[pl.ds(off, 16)], iv, mask=keep)
                    plsc.store_compressed(
                        wl_b.at[pl.ds(off, 16)], bv, mask=keep)

                return kcnt

            adv = lax.cond(fast, fastf, slowf, 0)
            return off + adv, seen + mcnt

        return lax.fori_loop(0, 64, chunk, carry)

    return lax.fori_loop(0, B // 1024, block, (0, 0))


def _split(wl_v, n, lo, sb_l, sb_m):
    sent = jnp.full((16,), jnp.int32(1 << 28), jnp.int32)

    def initc(t, carry):
        sb_l[pl.ds(t * 16, 16)] = sent
        return carry

    lax.fori_loop(0, SB * SBC // 16, initc, 0)

    def chunk(q, offs):
        lanes = wl_v[pl.ds(q * 16, 16)] - lo
        live = (q * 16 + _IOTA()) < n
        sub = jnp.clip(lanes >> 10, 0, SB - 1)
        mvec = q * 16 + _IOTA()
        out = []
        for s in range(SB):
            off_s = offs[s]
            msk = live & (sub == s)
            cnt = plsc.all_reduce_population_count(msk)[0]

            @pl.when((cnt > 0) & (off_s <= SBC - 16))
            def _(s=s, off_s=off_s, msk=msk):
                plsc.store_compressed(
                    sb_l.at[pl.ds(s * SBC + off_s, 16)], lanes, mask=msk)
                plsc.store_compressed(
                    sb_m.at[pl.ds(s * SBC + off_s, 16)], mvec, mask=msk)

            out.append(off_s + cnt)
        return tuple(out)

    offs = lax.fori_loop(0, (n + 15) // 16, chunk, (0,) * SB)
    ovf = offs[0] > (SBC - 16)
    for s in range(1, SB):
        ovf = ovf | (offs[s] > (SBC - 16))
    return ovf


def _stage1_body(user_h, itemi_h, itemj_h, eu_h, ei_h,
                 u_out, v_out,
                 idxbuf, slabs, tailbuf, ministage, blistm,
                 wl_vA, wl_bA, sb_lA, sb_mA, stageA, blistA, pend_lA, pend_mA,
                 wl_vB, wl_bB, sb_lB, sb_mB, stageB, blistB, pend_lB, pend_mB,
                 sem_i, sem_s, sem_oA, sem_oB, sem_m):
    w = lax.axis_index("s") * 2 + lax.axis_index("c")
    c0 = w * BCOLS
    lo = c0 * 128
    hi = jnp.where(w == 31, V, (c0 + BCOLS) * 128)
    ncols = BCOLS + jnp.where(w == 31, 4, 0)

    setA = (wl_vA, wl_bA, sb_lA, sb_mA, stageA, blistA, pend_lA, pend_mA,
            sem_oA)
    setB = (wl_vB, wl_bB, sb_lB, sb_mB, stageB, blistB, pend_lB, pend_mB,
            sem_oB)

    def run_pass(tab_hbm, lists):

        def issue_slab(s):
            r = lax.rem(s, NR)
            lane0 = pl.multiple_of((c0 + s) * 128, 128)
            pltpu.async_copy(
                tab_hbm.at[:, pl.ds(lane0, 128)],
                slabs.at[r], sem_s.at[r])

        def extract_chunk(rel, msk, r, stage_ref, rowbase):
            slot = jnp.full((16,), 0, jnp.int32) + r
            lin = rel & 127
            for c in range(D):
                col = jnp.full((16,), c, jnp.int32)
                vals = plsc.load_gather(slabs, [slot, col, lin], mask=msk)
                plsc.store_scatter(stage_ref, [rowbase + _IOTA(), col], vals,
                                   mask=msk)

        def round_body(carry):
            states = []
            flat = carry
            for li, (idx_hbm, out_hbm, roff, dummy, sset) in enumerate(lists):
                done = flat[2 * li]
                (wl_v, wl_b, sb_l, sb_m, stage, blist, pend_l, pend_m,
                 sem_o) = sset
                n, total = _bucket(idx_hbm, idxbuf, wl_v, wl_b, done, lo, hi,
                                   sem_i)
                ovf = _split(wl_v, n, lo, sb_l, sb_m)
                states.append((n, total, ovf))

            for kk in range(NR - 1):
                @pl.when(kk < ncols)
                def _(kk=kk):
                    issue_slab(jnp.int32(kk))

            def sbody(s, carry):
                @pl.when(s + NR - 1 < ncols)
                def _():
                    issue_slab(s + NR - 1)

                r = lax.rem(s, NR)
                par = lax.rem(s, 2)
                pltpu.make_async_copy(
                    tab_hbm.at[:, pl.ds(0, 128)],
                    slabs.at[r], sem_s.at[r]).wait()
                slab_l0 = s * 128
                sw = s >> 3

                for li, (idx_hbm, out_hbm, roff, dummy, sset) in enumerate(
                        lists):
                    (wl_v, wl_b, sb_l, sb_m, stage, blist, pend_l, pend_m,
                     sem_o) = sset
                    n, total, ovf = states[li]
                    dum = dummy

                    def pchunk(q, poff):
                        lanes = sb_l[pl.ds(sw * SBC + q * 16, 16)]
                        mv = sb_m[pl.ds(sw * SBC + q * 16, 16)]
                        rel = lanes - slab_l0
                        msk = (rel >= 0) & (rel < 128)
                        cnt = plsc.all_reduce_population_count(msk)[0]

                        @pl.when((cnt > 0) & (poff <= PC - 16))
                        def _():
                            plsc.store_compressed(
                                pend_l.at[pl.ds(poff, 16)], rel, mask=msk)
                            plsc.store_compressed(
                                pend_m.at[pl.ds(poff, 16)], mv, mask=msk)

                        return poff + cnt

                    poff = lax.fori_loop(0, SBC // 16, pchunk, 0)
                    use_slow = ovf | (poff > PC - 16)
                    neff = jnp.where(use_slow, 0, poff)

                    @pl.when(s >= 2)
                    def _():
                        pltpu.make_async_copy(
                            stage.at[par, pl.ds(0, PC)],
                            out_hbm.at[blist.at[par]], sem_o.at[par]).wait()

                    def fblock(pb, carry):
                        livemask = (pb * 16 + _IOTA()) < neff
                        rel = pend_l[pl.ds(pb * 16, 16)]
                        extract_chunk(rel, livemask, r,
                                      stage.at[par], pb * 16)
                        return carry

                    lax.fori_loop(0, (neff + 15) // 16, fblock, 0)

                    for q in range(PC // 16):
                        slotv = q * 16 + _IOTA()
                        mvq = pend_m[pl.ds(q * 16, 16)]
                        bq = plsc.load_gather(wl_b, [jnp.clip(mvq, 0, C + 15)])
                        rows = jnp.where(slotv < neff, bq + roff, dum)
                        blist[par, pl.ds(q * 16, 16)] = rows

                    pltpu.async_copy(
                        stage.at[par, pl.ds(0, PC)],
                        out_hbm.at[blist.at[par]], sem_o.at[par])

                    @pl.when(use_slow)
                    def _():
                        def sc(q, carry):
                            iv = wl_v[pl.ds(q * 16, 16)]
                            live = (q * 16 + _IOTA()) < n
                            rel = iv - lo - slab_l0
                            msk = live & (rel >= 0) & (rel < 128)
                            cnt = plsc.all_reduce_population_count(msk)[0]

                            @pl.when(cnt > 0)
                            def _():
                                extract_chunk(rel, msk, r, ministage, 0)
                                bq = wl_b[pl.ds(q * 16, 16)]
                                rowsm = jnp.where(msk, bq + roff, dum)
                                blistm[0, pl.ds(0, 16)] = rowsm
                                pltpu.async_copy(
                                    ministage.at[pl.ds(0, 16)],
                                    out_hbm.at[blistm.at[0]], sem_m).wait()

                            return carry

                        lax.fori_loop(0, (n + 15) // 16, sc, 0)

                return carry

            lax.fori_loop(0, ncols, sbody, 0)

            @pl.when(w == 31)
            def _():
                pltpu.async_copy(
                    tab_hbm.at[:, pl.ds((V // 128) * 128, 64)],
                    tailbuf, sem_s.at[0]).wait()
                tail_l0 = (V // 128) * 128 - lo
                for li, (idx_hbm, out_hbm, roff, dummy, sset) in enumerate(
                        lists):
                    (wl_v, wl_b, sb_l, sb_m, stage, blist, pend_l, pend_m,
                     sem_o) = sset
                    n, total, ovf = states[li]
                    dum = dummy

                    def tc(q, carry):
                        iv = wl_v[pl.ds(q * 16, 16)]
                        live = (q * 16 + _IOTA()) < n
                        rel = iv - lo - tail_l0
                        msk = live & (rel >= 0) & (rel < 64)
                        cnt = plsc.all_reduce_population_count(msk)[0]

                        @pl.when(cnt > 0)
                        def _():
                            lin = rel & 63
                            for c in range(D):
                                col = jnp.full((16,), c, jnp.int32)
                                vals = plsc.load_gather(
                                    tailbuf, [col, lin], mask=msk)
                                plsc.store_scatter(
                                    ministage, [_IOTA(), col], vals, mask=msk)
                            bq = wl_b[pl.ds(q * 16, 16)]
                            rowsm = jnp.where(msk, bq + roff, dum)
                            blistm[0, pl.ds(0, 16)] = rowsm
                            pltpu.async_copy(
                                ministage.at[pl.ds(0, 16)],
                                out_hbm.at[blistm.at[0]], sem_m).wait()

                        return carry

                    lax.fori_loop(0, (n + 15) // 16, tc, 0)

            out_carry = []
            for li, (idx_hbm, out_hbm, roff, dummy, sset) in enumerate(lists):
                (wl_v, wl_b, sb_l, sb_m, stage, blist, pend_l, pend_m,
                 sem_o) = sset
                for par in (0, 1):
                    pltpu.make_async_copy(
                        stage.at[par, pl.ds(0, PC)],
                        out_hbm.at[blist.at[par]], sem_o.at[par]).wait()
                n, total, ovf = states[li]
                done = flat[2 * li]
                out_carry.extend([done + n, total])
            return tuple(out_carry)

        def cond(carry):
            more = carry[0] < carry[1]
            for li in range(1, len(lists)):
                more = more | (carry[2 * li] < carry[2 * li + 1])
            return more

        lax.while_loop(cond, round_body, (0, 1) * len(lists))

    run_pass(eu_h, ((user_h, u_out, 0, B + w, setA),))
    run_pass(ei_h, ((itemi_h, v_out, 0, B + w, setA),
                    (itemj_h, v_out, JOFF, JOFF + B + w, setB)))


def _stage2_body(u_h, v_h, pi_h, pj_h, bu, bi, bj, po_i, po_j, sem):
    w = lax.axis_index("s") * 2 + lax.axis_index("c")
    base = w * BPW

    for ch in range(BPW // 128):
        cb = base + ch * 128
        cu = pltpu.async_copy(u_h.at[pl.ds(cb, 128)], bu, sem.at[0])
        ci = pltpu.async_copy(v_h.at[pl.ds(cb, 128)], bi, sem.at[1])
        cj = pltpu.async_copy(v_h.at[pl.ds(JOFF + cb, 128)], bj, sem.at[2])
        cu.wait()
        ci.wait()
        cj.wait()

        def group(g, carry):
            rows = g * 16 + _IOTA()
            acc_i = jnp.zeros((16,), jnp.float32)
            acc_j = jnp.zeros((16,), jnp.float32)
            for c in range(D):
                col = jnp.full((16,), c, jnp.int32)
                uc = plsc.load_gather(bu, [rows, col])
                vic = plsc.load_gather(bi, [rows, col])
                vjc = plsc.load_gather(bj, [rows, col])
                acc_i = acc_i + uc * vic
                acc_j = acc_j + uc * vjc
            po_i[pl.ds(ch * 128 + g * 16, 16)] = acc_i
            po_j[pl.ds(ch * 128 + g * 16, 16)] = acc_j
            return carry

        lax.fori_loop(0, 8, group, 0)

    pltpu.sync_copy(po_i, pi_h.at[pl.ds(base, BPW)])
    pltpu.sync_copy(po_j, pj_h.at[pl.ds(base, BPW)])


def kernel(user, item_i, item_j, embed_user, embed_item):
    mesh = plsc.VectorSubcoreMesh(core_axis_name="c", subcore_axis_name="s")
    eu_t = embed_user.T
    ei_t = embed_item.T

    def listset():
        return [
            pltpu.VMEM((C + 32,), jnp.int32),
            pltpu.VMEM((C + 32,), jnp.int32),
            pltpu.VMEM((SB * SBC,), jnp.int32),
            pltpu.VMEM((SB * SBC,), jnp.int32),
            pltpu.VMEM((2, PC, 128), jnp.float32),
            pltpu.VMEM((2, PC), jnp.int32),
            pltpu.VMEM((SBC + 16,), jnp.int32),
            pltpu.VMEM((SBC + 16,), jnp.int32),
        ]

    stage1 = pl.kernel(
        _stage1_body,
        mesh=mesh,
        compiler_params=pltpu.CompilerParams(
            needs_layout_passes=False, use_tc_tiling_on_sc=True),
        out_type=(
            jax.ShapeDtypeStruct((UROWS, 128), jnp.float32),
            jax.ShapeDtypeStruct((VROWS, 128), jnp.float32),
        ),
        scratch_types=[
            pltpu.VMEM((1024,), jnp.int32),
            pltpu.VMEM((NR, D, 128), jnp.float32),
            pltpu.VMEM((D, 64), jnp.float32),
            pltpu.VMEM((16, 128), jnp.float32),
            pltpu.VMEM((1, 16), jnp.int32),
        ] + listset() + listset() + [
            pltpu.SemaphoreType.DMA,
            pltpu.SemaphoreType.DMA((NR,)),
            pltpu.SemaphoreType.DMA((2,)),
            pltpu.SemaphoreType.DMA((2,)),
            pltpu.SemaphoreType.DMA,
        ],
    )

    u_rows, v_rows = stage1(user, item_i, item_j, eu_t, ei_t)

    stage2 = pl.kernel(
        _stage2_body,
        mesh=mesh,
        compiler_params=pltpu.CompilerParams(
            needs_layout_passes=False, use_tc_tiling_on_sc=True),
        out_type=(
            jax.ShapeDtypeStruct((B,), jnp.float32),
            jax.ShapeDtypeStruct((B,), jnp.float32),
        ),
        scratch_types=[
            pltpu.VMEM((128, 128), jnp.float32),
            pltpu.VMEM((128, 128), jnp.float32),
            pltpu.VMEM((128, 128), jnp.float32),
            pltpu.VMEM((BPW,), jnp.float32),
            pltpu.VMEM((BPW,), jnp.float32),
            pltpu.SemaphoreType.DMA((3,)),
        ],
    )
    return stage2(u_rows, v_rows)


# speedup vs baseline: 2.9224x; 2.9224x over previous
import jax
import jax.numpy as jnp
from jax import lax
from jax.experimental import pallas as pl
from jax.experimental.pallas import tpu as pltpu
from jax.experimental.pallas import tpu_sc as plsc

B = 16384
D = 64
V = 1000000
NW = 32
BPW = B // NW
BCOLS = 244
NR = 8
C = 1024
SB = 32
SBC = 64
PC = 32
SLOTS = 81920
DV = 1 << 28

_IOTA = lambda: lax.iota(jnp.int32, 16)


def _bucket(idx_hbm, idxbuf, wl_v, wl_b, done, lo, hi, sem):
    def block(blk, carry):
        pltpu.async_copy(
            idx_hbm.at[pl.ds(blk * 1024, 1024)], idxbuf, sem).wait()

        def chunk(k, carry):
            off, seen = carry
            iv = idxbuf[pl.ds(k * 16, 16)]
            bv = blk * 1024 + k * 16 + _IOTA()
            m = (iv >= lo) & (iv < hi)
            mcnt = plsc.all_reduce_population_count(m)[0]
            fast = (done == 0) & (off + 16 <= C)

            def fastf(_):
                @pl.when(mcnt > 0)
                def _():
                    plsc.store_compressed(
                        wl_v.at[pl.ds(off, 16)], iv, mask=m)
                    plsc.store_compressed(
                        wl_b.at[pl.ds(off, 16)], bv, mask=m)

                return mcnt

            def slowf(_):
                rank = seen + plsc.cumsum(jnp.where(m, 1, 0))
                keep = m & (rank > done) & (rank <= done + C)
                kcnt = plsc.all_reduce_population_count(keep)[0]

                @pl.when(kcnt > 0)
                def _():
                    plsc.store_compressed(
                        wl_v.at[pl.ds(off, 16)], iv, mask=keep)
                    plsc.store_compressed(
                        wl_b.at[pl.ds(off, 16)], bv, mask=keep)

                return kcnt

            adv = lax.cond(fast, fastf, slowf, 0)
            return off + adv, seen + mcnt

        return lax.fori_loop(0, 64, chunk, carry)

    return lax.fori_loop(0, B // 1024, block, (0, 0))


def _split(wl_v, n, lo, sb_l, sb_m):
    sent = jnp.full((16,), jnp.int32(DV), jnp.int32)

    def initc(t, carry):
        sb_l[pl.ds(t * 16, 16)] = sent
        return carry

    lax.fori_loop(0, SB * SBC // 16, initc, 0)

    def chunk(q, offs):
        lanes = wl_v[pl.ds(q * 16, 16)] - lo
        live = (q * 16 + _IOTA()) < n
        sub = jnp.clip(lanes >> 10, 0, SB - 1)
        mvec = q * 16 + _IOTA()
        out = []
        for s in range(SB):
            off_s = offs[s]
            msk = live & (sub == s)
            cnt = plsc.all_reduce_population_count(msk)[0]

            @pl.when((cnt > 0) & (off_s <= SBC - 16))
            def _(s=s, off_s=off_s, msk=msk):
                plsc.store_compressed(
                    sb_l.at[pl.ds(s * SBC + off_s, 16)], lanes, mask=msk)
                plsc.store_compressed(
                    sb_m.at[pl.ds(s * SBC + off_s, 16)], mvec, mask=msk)

            out.append(off_s + cnt)
        return tuple(out)

    offs = lax.fori_loop(0, (n + 15) // 16, chunk, (0,) * SB)
    ovf = offs[0] > (SBC - 16)
    for s in range(1, SB):
        ovf = ovf | (offs[s] > (SBC - 16))
    return ovf


def _stage1_body(user_h, itemi_h, itemj_h, eu_h, ei_h,
                 dense_out, bmap_out, counts_out,
                 idxbuf, slabs, tailbuf, stage, stage_b,
                 wl_vA, wl_bA, sb_lA, sb_mA, pend_lA, pend_mA,
                 wl_vB, wl_bB, sb_lB, sb_mB, pend_lB, pend_mB,
                 sem_i, sem_s, sem_f):
    w = lax.axis_index("s") * 2 + lax.axis_index("c")
    c0 = w * BCOLS
    lo = c0 * 128
    hi = jnp.where(w == 31, V, (c0 + BCOLS) * 128)
    ncols = BCOLS + jnp.where(w == 31, 4, 0)
    dvv = jnp.full((16,), jnp.int32(DV), jnp.int32)

    setA = (wl_vA, wl_bA, sb_lA, sb_mA, pend_lA, pend_mA)
    setB = (wl_vB, wl_bB, sb_lB, sb_mB, pend_lB, pend_mB)

    def init_stage_b(par):
        def ic(t, carry):
            stage_b[pl.ds(par * 128 + t * 16, 16)] = dvv
            return carry

        lax.fori_loop(0, 8, ic, 0)

    def flush(nf):
        par = lax.rem(nf, 2)
        rowoff = w * SLOTS + nf * 128
        c1 = pltpu.async_copy(
            stage.at[par], dense_out.at[pl.ds(rowoff, 128)], sem_f.at[par])
        c2 = pltpu.async_copy(
            stage_b.at[pl.ds(par * 128, 128)],
            bmap_out.at[pl.ds(rowoff, 128)], sem_f.at[par])
        c1.wait()
        c2.wait()
        init_stage_b(lax.rem(nf + 1, 2))

    def append16(tail_src, rvec, lin, msk, livecnt, bvals, soff, nf):
        need = (soff + 16) > 128

        @pl.when(need)
        def _():
            flush(nf)

        soff2 = jnp.where(need, 0, soff)
        nf2 = nf + jnp.where(need, 1, 0)
        par = lax.rem(nf2, 2)
        pos = plsc.cumsum(jnp.where(msk, 1, 0)) - 1
        rowvec = jnp.clip(soff2 + pos, 0, 127)
        for c in range(D):
            col = jnp.full((16,), c, jnp.int32)
            if tail_src is None:
                vals = plsc.load_gather(slabs, [rvec, col, lin], mask=msk)
            else:
                vals = plsc.load_gather(tail_src, [col, lin], mask=msk)
            plsc.store_scatter(stage.at[par], [rowvec, col], vals, mask=msk)
        plsc.store_scatter(stage_b, [par * 128 + rowvec], bvals, mask=msk)
        return soff2 + livecnt, nf2

    def run_pass(tab_hbm, lists, fstate):
        nl = len(lists)

        def issue_slab(s):
            r = lax.rem(s, NR)
            lane0 = pl.multiple_of((c0 + s) * 128, 128)
            pltpu.async_copy(
                tab_hbm.at[:, pl.ds(lane0, 128)],
                slabs.at[r], sem_s.at[r])

        def round_body(carry):
            fc = (carry[2 * nl], carry[2 * nl + 1])
            states = []
            for li, (idx_hbm, roff, sset) in enumerate(lists):
                done = carry[2 * li]
                (wl_v, wl_b, sb_l, sb_m, pend_l, pend_m) = sset
                n, total = _bucket(idx_hbm, idxbuf, wl_v, wl_b, done, lo, hi,
                                   sem_i)
                ovf = _split(wl_v, n, lo, sb_l, sb_m)
                states.append((n, total, ovf))

            for kk in range(NR - 1):
                @pl.when(kk < ncols)
                def _(kk=kk):
                    issue_slab(jnp.int32(kk))

            def sbody(s, fc2):
                soff, nf = fc2

                @pl.when(s + NR - 1 < ncols)
                def _():
                    issue_slab(s + NR - 1)

                r = lax.rem(s, NR)
                pltpu.make_async_copy(
                    tab_hbm.at[:, pl.ds(0, 128)],
                    slabs.at[r], sem_s.at[r]).wait()
                slab_l0 = s * 128
                sw = s >> 3
                rvec = jnp.full((16,), 0, jnp.int32) + r

                for li, (idx_hbm, roff, sset) in enumerate(lists):
                    (wl_v, wl_b, sb_l, sb_m, pend_l, pend_m) = sset
                    n, total, ovf = states[li]

                    def pchunk(q, poff):
                        lanes = sb_l[pl.ds(sw * SBC + q * 16, 16)]
                        mv = sb_m[pl.ds(sw * SBC + q * 16, 16)]
                        rel = lanes - slab_l0
                        msk = (rel >= 0) & (rel < 128)
                        cnt = plsc.all_reduce_population_count(msk)[0]

                        @pl.when((cnt > 0) & (poff <= PC - 16))
                        def _():
                            plsc.store_compressed(
                                pend_l.at[pl.ds(poff, 16)], rel, mask=msk)
                            plsc.store_compressed(
                                pend_m.at[pl.ds(poff, 16)], mv, mask=msk)

                        return poff + cnt

                    poff = lax.fori_loop(0, SBC // 16, pchunk, 0)
                    use_slow = ovf | (poff > PC - 16)
                    neff = jnp.where(use_slow, 0, poff)

                    def fblock(pb, fc3):
                        soff3, nf3 = fc3
                        left = neff - pb * 16
                        livemask = _IOTA() < left
                        livecnt = jnp.minimum(left, 16)
                        rel = pend_l[pl.ds(pb * 16, 16)]
                        mvq = pend_m[pl.ds(pb * 16, 16)]
                        bq = plsc.load_gather(
                            wl_b, [jnp.clip(mvq, 0, C + 15)]) + roff
                        return append16(None, rvec, rel & 127, livemask,
                                        livecnt, bq, soff3, nf3)

                    soff, nf = lax.fori_loop(0, (neff + 15) // 16, fblock,
                                             (soff, nf))

                    def sc(q, fc3):
                        soff3, nf3 = fc3
                        iv = wl_v[pl.ds(q * 16, 16)]
                        live = (q * 16 + _IOTA()) < n
                        rel = iv - lo - slab_l0
                        msk = live & (rel >= 0) & (rel < 128)
                        cnt = plsc.all_reduce_population_count(msk)[0]

                        def go(fc4):
                            soff4, nf4 = fc4
                            bq = wl_b[pl.ds(q * 16, 16)] + roff
                            return append16(None, rvec, rel & 127, msk,
                                            cnt, bq, soff4, nf4)

                        return lax.cond(cnt > 0, go, lambda fc4: fc4,
                                        (soff3, nf3))

                    nsc = jnp.where(use_slow, (n + 15) // 16, 0)
                    soff, nf = lax.fori_loop(0, nsc, sc, (soff, nf))

                return soff, nf

            fc = lax.fori_loop(0, ncols, sbody, fc)

            def tailf(fc2):
                pltpu.async_copy(
                    tab_hbm.at[:, pl.ds((V // 128) * 128, 64)],
                    tailbuf, sem_s.at[0]).wait()
                tail_l0 = (V // 128) * 128 - lo
                for li, (idx_hbm, roff, sset) in enumerate(lists):
                    (wl_v, wl_b, sb_l, sb_m, pend_l, pend_m) = sset
                    n, total, ovf = states[li]

                    def tc(q, fc3):
                        soff3, nf3 = fc3
                        iv = wl_v[pl.ds(q * 16, 16)]
                        live = (q * 16 + _IOTA()) < n
                        rel = iv - lo - tail_l0
                        msk = live & (rel >= 0) & (rel < 64)
                        cnt = plsc.all_reduce_population_count(msk)[0]

                        def go(fc4):
                            soff4, nf4 = fc4
                            bq = wl_b[pl.ds(q * 16, 16)] + roff
                            return append16(tailbuf, None, rel & 63, msk,
                                            cnt, bq, soff4, nf4)

                        return lax.cond(cnt > 0, go, lambda fc4: fc4,
                                        (soff3, nf3))

                    fc2 = lax.fori_loop(0, (n + 15) // 16, tc, fc2)
                return fc2

            fc = lax.cond(w == 31, tailf, lambda fc2: fc2, fc)

            out_carry = []
            for li in range(nl):
                n, total, ovf = states[li]
                out_carry.extend([carry[2 * li] + n, total])
            return tuple(out_carry) + fc

        def cond(carry):
            more = carry[0] < carry[1]
            for li in range(1, nl):
                more = more | (carry[2 * li] < carry[2 * li + 1])
            return more

        fin = lax.while_loop(cond, round_body,
                             tuple([0, 1] * nl) + tuple(fstate))
        return fin[2 * nl], fin[2 * nl + 1]

    init_stage_b(0)
    fstate = (0, 0)
    fstate = run_pass(eu_h, ((user_h, 0, setA),), fstate)
    fstate = run_pass(ei_h, ((itemi_h, B, setA),
                             (itemj_h, 2 * B, setB)), fstate)
    soff, nf = fstate

    @pl.when(soff > 0)
    def _():
        flush(nf)

    nf_tot = nf + jnp.where(soff > 0, 1, 0)

    cv = jnp.where(_IOTA() == 0, nf_tot * 128, 0)
    pend_lA[pl.ds(0, 16)] = cv
    pltpu.sync_copy(pend_lA.at[pl.ds(0, 16)],
                    counts_out.at[pl.ds(w * 16, 16)])


def _stage2_body(dense_h, bmap_h, counts_h, pi_h, pj_h,
                 cnt_v, bmbuf, pos_all,
                 gu, gi, gj, po_i, po_j, sem):
    w = lax.axis_index("s") * 2 + lax.axis_index("c")
    base = w * BPW
    SBK = 4096

    pltpu.sync_copy(counts_h, cnt_v)

    zv = jnp.zeros((16,), jnp.int32)

    def pinit(t, carry):
        pos_all[pl.ds(t * 16, 16)] = zv
        return carry

    lax.fori_loop(0, 3 * BPW // 16, pinit, 0)

    def scan_sb(pbase, sw, sb0, csw):
        def chunk(k, carry):
            bv = bmbuf[pl.ds(pbase + k * 16, 16)]
            live = (sb0 + k * 16 + _IOTA()) < csw
            live = live & (bv < jnp.int32(3 * B))
            reg = jnp.clip(bv >> 14, 0, 2)
            b = bv & (B - 1)
            mine = live & (b >= base) & (b < base + BPW)
            slotv = sw * SLOTS + sb0 + k * 16 + _IOTA()
            rowl = reg * BPW + (b - base)
            plsc.store_scatter(pos_all, [rowl], slotv, mask=mine)
            return carry

        lax.fori_loop(0, SBK // 16, chunk, 0)

    def issue_sb(sw, par):
        pltpu.async_copy(
            bmap_h.at[pl.ds(sw * SLOTS, SBK)],
            bmbuf.at[pl.ds(par * SBK, SBK)], sem.at[3 + par])

    issue_sb(0, 0)
    for sw in range(NW):
        par = sw & 1
        pltpu.make_async_copy(
            bmap_h.at[pl.ds(0, SBK)],
            bmbuf.at[pl.ds(par * SBK, SBK)], sem.at[3 + par]).wait()
        if sw + 1 < NW:
            issue_sb(sw + 1, (sw + 1) & 1)
        csw = jnp.max(cnt_v[pl.ds(sw * 16, 16)])
        scan_sb(par * SBK, sw, 0, csw)

        def sbf(sb, carry):
            pltpu.async_copy(
                bmap_h.at[pl.ds(sw * SLOTS + (sb + 1) * SBK, SBK)],
                bmbuf.at[pl.ds(par * SBK, SBK)], sem.at[3 + par]).wait()
            scan_sb(par * SBK, sw, (sb + 1) * SBK, csw)
            return carry

        lax.fori_loop(0, (csw + SBK - 1) // SBK - 1, sbf, 0)

    for ch in range(BPW // 128):
        cu = pltpu.async_copy(
            dense_h.at[pos_all.at[pl.ds(ch * 128, 128)]], gu, sem.at[0])
        ci = pltpu.async_copy(
            dense_h.at[pos_all.at[pl.ds(BPW + ch * 128, 128)]], gi,
            sem.at[1])
        cj = pltpu.async_copy(
            dense_h.at[pos_all.at[pl.ds(2 * BPW + ch * 128, 128)]], gj,
            sem.at[2])
        cu.wait()
        ci.wait()
        cj.wait()

        def group(g, carry):
            rows = g * 16 + _IOTA()
            acc_i = jnp.zeros((16,), jnp.float32)
            acc_j = jnp.zeros((16,), jnp.float32)
            for c in range(D):
                col = jnp.full((16,), c, jnp.int32)
                uc = plsc.load_gather(gu, [rows, col])
                vic = plsc.load_gather(gi, [rows, col])
                vjc = plsc.load_gather(gj, [rows, col])
                acc_i = acc_i + uc * vic
                acc_j = acc_j + uc * vjc
            po_i[pl.ds(ch * 128 + g * 16, 16)] = acc_i
            po_j[pl.ds(ch * 128 + g * 16, 16)] = acc_j
            return carry

        lax.fori_loop(0, 8, group, 0)

    pltpu.sync_copy(po_i, pi_h.at[pl.ds(base, BPW)])
    pltpu.sync_copy(po_j, pj_h.at[pl.ds(base, BPW)])


def kernel(user, item_i, item_j, embed_user, embed_item):
    mesh = plsc.VectorSubcoreMesh(core_axis_name="c", subcore_axis_name="s")
    eu_t = embed_user.T
    ei_t = embed_item.T

    def listset():
        return [
            pltpu.VMEM((C + 32,), jnp.int32),
            pltpu.VMEM((C + 32,), jnp.int32),
            pltpu.VMEM((SB * SBC,), jnp.int32),
            pltpu.VMEM((SB * SBC,), jnp.int32),
            pltpu.VMEM((SBC + 16,), jnp.int32),
            pltpu.VMEM((SBC + 16,), jnp.int32),
        ]

    stage1 = pl.kernel(
        _stage1_body,
        mesh=mesh,
        compiler_params=pltpu.CompilerParams(
            needs_layout_passes=False, use_tc_tiling_on_sc=True),
        out_type=(
            jax.ShapeDtypeStruct((NW * SLOTS, 128), jnp.float32),
            jax.ShapeDtypeStruct((NW * SLOTS,), jnp.int32),
            jax.ShapeDtypeStruct((NW * 16,), jnp.int32),
        ),
        scratch_types=[
            pltpu.VMEM((1024,), jnp.int32),
            pltpu.VMEM((NR, D, 128), jnp.float32),
            pltpu.VMEM((D, 64), jnp.float32),
            pltpu.VMEM((2, 128, 128), jnp.float32),
            pltpu.VMEM((256,), jnp.int32),
        ] + listset() + listset() + [
            pltpu.SemaphoreType.DMA,
            pltpu.SemaphoreType.DMA((NR,)),
            pltpu.SemaphoreType.DMA((2,)),
        ],
    )
    dense, bmap, counts = stage1(user, item_i, item_j, eu_t, ei_t)

    stage2 = pl.kernel(
        _stage2_body,
        mesh=mesh,
        compiler_params=pltpu.CompilerParams(
            needs_layout_passes=False, use_tc_tiling_on_sc=True),
        out_type=(
            jax.ShapeDtypeStruct((B,), jnp.float32),
            jax.ShapeDtypeStruct((B,), jnp.float32),
        ),
        scratch_types=[
            pltpu.VMEM((NW * 16,), jnp.int32),
            pltpu.VMEM((2 * 4096,), jnp.int32),
            pltpu.VMEM((3 * BPW,), jnp.int32),
            pltpu.VMEM((128, 128), jnp.float32),
            pltpu.VMEM((128, 128), jnp.float32),
            pltpu.VMEM((128, 128), jnp.float32),
            pltpu.VMEM((BPW,), jnp.float32),
            pltpu.VMEM((BPW,), jnp.float32),
            pltpu.SemaphoreType.DMA((5,)),
        ],
    )
    return stage2(dense, bmap, counts)


# speedup vs baseline: 3.6633x; 1.2535x over previous
import jax
import jax.numpy as jnp
from jax import lax
from jax.experimental import pallas as pl
from jax.experimental.pallas import tpu as pltpu
from jax.experimental.pallas import tpu_sc as plsc

B = 16384
D = 64
V = 1000000
NW = 32
BPW = B // NW
BCOLS = 244
NR = 4
SLW = 256
C = 1024
SB = 32
SBC = 64
PC = 32
SLOTS = 81920
DV = 1 << 28

_IOTA = lambda: lax.iota(jnp.int32, 16)


def _bucket(idx_hbm, idxbuf, wl_v, wl_b, done, lo, hi, sem):
    def block(blk, carry):
        pltpu.async_copy(
            idx_hbm.at[pl.ds(blk * 1024, 1024)], idxbuf, sem).wait()

        def chunk(k, carry):
            off, seen = carry
            iv = idxbuf[pl.ds(k * 16, 16)]
            bv = blk * 1024 + k * 16 + _IOTA()
            m = (iv >= lo) & (iv < hi)
            mcnt = plsc.all_reduce_population_count(m)[0]
            fast = (done == 0) & (off + 16 <= C)

            def fastf(_):
                @pl.when(mcnt > 0)
                def _():
                    plsc.store_compressed(
                        wl_v.at[pl.ds(off, 16)], iv, mask=m)
                    plsc.store_compressed(
                        wl_b.at[pl.ds(off, 16)], bv, mask=m)

                return mcnt

            def slowf(_):
                rank = seen + plsc.cumsum(jnp.where(m, 1, 0))
                keep = m & (rank > done) & (rank <= done + C)
                kcnt = plsc.all_reduce_population_count(keep)[0]

                @pl.when(kcnt > 0)
                def _():
                    plsc.store_compressed(
                        wl_v.at[pl.ds(off, 16)], iv, mask=keep)
                    plsc.store_compressed(
                        wl_b.at[pl.ds(off, 16)], bv, mask=keep)

                return kcnt

            adv = lax.cond(fast, fastf, slowf, 0)
            return off + adv, seen + mcnt

        return lax.fori_loop(0, 64, chunk, carry)

    return lax.fori_loop(0, B // 1024, block, (0, 0))


def _split(wl_v, n, lo, sb_l, sb_m):
    sent = jnp.full((16,), jnp.int32(DV), jnp.int32)

    def initc(t, carry):
        sb_l[pl.ds(t * 16, 16)] = sent
        return carry

    lax.fori_loop(0, SB * SBC // 16, initc, 0)

    def chunk(q, offs):
        lanes = wl_v[pl.ds(q * 16, 16)] - lo
        live = (q * 16 + _IOTA()) < n
        sub = jnp.clip(lanes >> 10, 0, SB - 1)
        mvec = q * 16 + _IOTA()
        out = []
        for s in range(SB):
            off_s = offs[s]
            msk = live & (sub == s)
            cnt = plsc.all_reduce_population_count(msk)[0]

            @pl.when((cnt > 0) & (off_s <= SBC - 16))
            def _(s=s, off_s=off_s, msk=msk):
                plsc.store_compressed(
                    sb_l.at[pl.ds(s * SBC + off_s, 16)], lanes, mask=msk)
                plsc.store_compressed(
                    sb_m.at[pl.ds(s * SBC + off_s, 16)], mvec, mask=msk)

            out.append(off_s + cnt)
        return tuple(out)

    offs = lax.fori_loop(0, (n + 15) // 16, chunk, (0,) * SB)
    ovf = offs[0] > (SBC - 16)
    for s in range(1, SB):
        ovf = ovf | (offs[s] > (SBC - 16))
    return ovf


def _stage1_body(user_h, itemi_h, itemj_h, eu_h, ei_h,
                 dense_out, bmap_out, counts_out,
                 idxbuf, slabs, tailbuf, stage, stage_b,
                 wl_vA, wl_bA, sb_lA, sb_mA, pend_lA, pend_mA,
                 wl_vB, wl_bB, sb_lB, sb_mB, pend_lB, pend_mB,
                 sem_i, sem_s, sem_f):
    w = lax.axis_index("s") * 2 + lax.axis_index("c")
    c0 = w * BCOLS
    lo = c0 * 128
    hi = jnp.where(w == 31, V, (c0 + BCOLS) * 128)
    ncols = BCOLS + jnp.where(w == 31, 4, 0)
    nslab = ncols * 128 // SLW
    dvv = jnp.full((16,), jnp.int32(DV), jnp.int32)

    setA = (wl_vA, wl_bA, sb_lA, sb_mA, pend_lA, pend_mA)
    setB = (wl_vB, wl_bB, sb_lB, sb_mB, pend_lB, pend_mB)

    def init_stage_b(par):
        def ic(t, carry):
            stage_b[pl.ds(par * 128 + t * 16, 16)] = dvv
            return carry

        lax.fori_loop(0, 8, ic, 0)

    def flush(nf):
        par = lax.rem(nf, 2)
        rowoff = w * SLOTS + nf * 128
        c1 = pltpu.async_copy(
            stage.at[par], dense_out.at[pl.ds(rowoff, 128)], sem_f.at[par])
        c2 = pltpu.async_copy(
            stage_b.at[pl.ds(par * 128, 128)],
            bmap_out.at[pl.ds(rowoff, 128)], sem_f.at[par])
        c1.wait()
        c2.wait()
        init_stage_b(lax.rem(nf + 1, 2))

    def append16(tail_src, rvec, lin, msk, livecnt, bvals, soff, nf):
        need = (soff + 16) > 128

        @pl.when(need)
        def _():
            flush(nf)

        soff2 = jnp.where(need, 0, soff)
        nf2 = nf + jnp.where(need, 1, 0)
        par = lax.rem(nf2, 2)
        pos = plsc.cumsum(jnp.where(msk, 1, 0)) - 1
        rowvec = jnp.clip(soff2 + pos, 0, 127)
        for c in range(D):
            col = jnp.full((16,), c, jnp.int32)
            if tail_src is None:
                vals = plsc.load_gather(slabs, [rvec, col, lin], mask=msk)
            else:
                vals = plsc.load_gather(tail_src, [col, lin], mask=msk)
            plsc.store_scatter(stage.at[par], [rowvec, col], vals, mask=msk)
        plsc.store_scatter(stage_b, [par * 128 + rowvec], bvals, mask=msk)
        return soff2 + livecnt, nf2

    def run_pass(tab_hbm, lists, fstate):
        nl = len(lists)

        def issue_slab(s):
            r = lax.rem(s, NR)
            lane0 = pl.multiple_of(c0 * 128 + s * SLW, 128)
            pltpu.async_copy(
                tab_hbm.at[:, pl.ds(lane0, SLW)],
                slabs.at[r], sem_s.at[r])

        def round_body(carry):
            fc = (carry[2 * nl], carry[2 * nl + 1])
            states = []
            for li, (idx_hbm, roff, sset) in enumerate(lists):
                done = carry[2 * li]
                (wl_v, wl_b, sb_l, sb_m, pend_l, pend_m) = sset
                n, total = _bucket(idx_hbm, idxbuf, wl_v, wl_b, done, lo, hi,
                                   sem_i)
                ovf = _split(wl_v, n, lo, sb_l, sb_m)
                states.append((n, total, ovf))

            for kk in range(NR - 1):
                @pl.when(kk < nslab)
                def _(kk=kk):
                    issue_slab(jnp.int32(kk))

            def sbody(s, fc2):
                soff, nf = fc2

                @pl.when(s + NR - 1 < nslab)
                def _():
                    issue_slab(s + NR - 1)

                r = lax.rem(s, NR)
                pltpu.make_async_copy(
                    tab_hbm.at[:, pl.ds(0, SLW)],
                    slabs.at[r], sem_s.at[r]).wait()
                slab_l0 = s * SLW
                sw = s >> 2
                rvec = jnp.full((16,), 0, jnp.int32) + r

                for li, (idx_hbm, roff, sset) in enumerate(lists):
                    (wl_v, wl_b, sb_l, sb_m, pend_l, pend_m) = sset
                    n, total, ovf = states[li]

                    def pchunk(q, poff):
                        lanes = sb_l[pl.ds(sw * SBC + q * 16, 16)]
                        mv = sb_m[pl.ds(sw * SBC + q * 16, 16)]
                        rel = lanes - slab_l0
                        msk = (rel >= 0) & (rel < SLW)
                        cnt = plsc.all_reduce_population_count(msk)[0]

                        @pl.when((cnt > 0) & (poff <= PC - 16))
                        def _():
                            plsc.store_compressed(
                                pend_l.at[pl.ds(poff, 16)], rel, mask=msk)
                            plsc.store_compressed(
                                pend_m.at[pl.ds(poff, 16)], mv, mask=msk)

                        return poff + cnt

                    poff = lax.fori_loop(0, SBC // 16, pchunk, 0)
                    use_slow = ovf | (poff > PC - 16)
                    neff = jnp.where(use_slow, 0, poff)

                    def fblock(pb, fc3):
                        soff3, nf3 = fc3
                        left = neff - pb * 16
                        livemask = _IOTA() < left
                        livecnt = jnp.minimum(left, 16)
                        rel = pend_l[pl.ds(pb * 16, 16)]
                        mvq = pend_m[pl.ds(pb * 16, 16)]
                        bq = plsc.load_gather(
                            wl_b, [jnp.clip(mvq, 0, C + 15)]) + roff
                        return append16(None, rvec, rel & (SLW - 1),
                                        livemask, livecnt, bq, soff3, nf3)

                    soff, nf = lax.fori_loop(0, (neff + 15) // 16, fblock,
                                             (soff, nf))

                    def sc(q, fc3):
                        soff3, nf3 = fc3
                        iv = wl_v[pl.ds(q * 16, 16)]
                        live = (q * 16 + _IOTA()) < n
                        rel = iv - lo - slab_l0
                        msk = live & (rel >= 0) & (rel < SLW)
                        cnt = plsc.all_reduce_population_count(msk)[0]

                        def go(fc4):
                            soff4, nf4 = fc4
                            bq = wl_b[pl.ds(q * 16, 16)] + roff
                            return append16(None, rvec, rel & (SLW - 1), msk,
                                            cnt, bq, soff4, nf4)

                        return lax.cond(cnt > 0, go, lambda fc4: fc4,
                                        (soff3, nf3))

                    nsc = jnp.where(use_slow, (n + 15) // 16, 0)
                    soff, nf = lax.fori_loop(0, nsc, sc, (soff, nf))

                return soff, nf

            fc = lax.fori_loop(0, nslab, sbody, fc)

            def tailf(fc2):
                pltpu.async_copy(
                    tab_hbm.at[:, pl.ds((V // 128) * 128, 64)],
                    tailbuf, sem_s.at[0]).wait()
                tail_l0 = (V // 128) * 128 - lo
                for li, (idx_hbm, roff, sset) in enumerate(lists):
                    (wl_v, wl_b, sb_l, sb_m, pend_l, pend_m) = sset
                    n, total, ovf = states[li]

                    def tc(q, fc3):
                        soff3, nf3 = fc3
                        iv = wl_v[pl.ds(q * 16, 16)]
                        live = (q * 16 + _IOTA()) < n
                        rel = iv - lo - tail_l0
                        msk = live & (rel >= 0) & (rel < 64)
                        cnt = plsc.all_reduce_population_count(msk)[0]

                        def go(fc4):
                            soff4, nf4 = fc4
                            bq = wl_b[pl.ds(q * 16, 16)] + roff
                            return append16(tailbuf, None, rel & 63, msk,
                                            cnt, bq, soff4, nf4)

                        return lax.cond(cnt > 0, go, lambda fc4: fc4,
                                        (soff3, nf3))

                    fc2 = lax.fori_loop(0, (n + 15) // 16, tc, fc2)
                return fc2

            fc = lax.cond(w == 31, tailf, lambda fc2: fc2, fc)

            out_carry = []
            for li in range(nl):
                n, total, ovf = states[li]
                out_carry.extend([carry[2 * li] + n, total])
            return tuple(out_carry) + fc

        def cond(carry):
            more = carry[0] < carry[1]
            for li in range(1, nl):
                more = more | (carry[2 * li] < carry[2 * li + 1])
            return more

        fin = lax.while_loop(cond, round_body,
                             tuple([0, 1] * nl) + tuple(fstate))
        return fin[2 * nl], fin[2 * nl + 1]

    init_stage_b(0)
    fstate = (0, 0)
    fstate = run_pass(eu_h, ((user_h, 0, setA),), fstate)
    fstate = run_pass(ei_h, ((itemi_h, B, setA),
                             (itemj_h, 2 * B, setB)), fstate)
    soff, nf = fstate

    @pl.when(soff > 0)
    def _():
        flush(nf)

    nf_tot = nf + jnp.where(soff > 0, 1, 0)

    cv = jnp.where(_IOTA() == 0, nf_tot * 128, 0)
    pend_lA[pl.ds(0, 16)] = cv
    pltpu.sync_copy(pend_lA.at[pl.ds(0, 16)],
                    counts_out.at[pl.ds(w * 16, 16)])


def _stage2_body(dense_h, bmap_h, counts_h, pi_h, pj_h,
                 cnt_v, bmbuf, pos_all,
                 gu, gi, gj, po_i, po_j, sem):
    w = lax.axis_index("s") * 2 + lax.axis_index("c")
    base = w * BPW
    SBK = 4096

    pltpu.sync_copy(counts_h, cnt_v)

    zv = jnp.zeros((16,), jnp.int32)

    def pinit(t, carry):
        pos_all[pl.ds(t * 16, 16)] = zv
        return carry

    lax.fori_loop(0, 3 * BPW // 16, pinit, 0)

    def scan_sb(pbase, sw, sb0, csw):
        def chunk(k, carry):
            bv = bmbuf[pl.ds(pbase + k * 16, 16)]
            live = (sb0 + k * 16 + _IOTA()) < csw
            live = live & (bv < jnp.int32(3 * B))
            reg = jnp.clip(bv >> 14, 0, 2)
            b = bv & (B - 1)
            mine = live & (b >= base) & (b < base + BPW)
            slotv = sw * SLOTS + sb0 + k * 16 + _IOTA()
            rowl = reg * BPW + (b - base)
            plsc.store_scatter(pos_all, [rowl], slotv, mask=mine)
            return carry

        nch = (jnp.clip(csw - sb0, 0, SBK) + 15) // 16
        lax.fori_loop(0, nch, chunk, 0)

    def issue_sb(sw, par):
        pltpu.async_copy(
            bmap_h.at[pl.ds(sw * SLOTS, SBK)],
            bmbuf.at[pl.ds(par * SBK, SBK)], sem.at[3 + par])

    issue_sb(0, 0)
    for sw in range(NW):
        par = sw & 1
        pltpu.make_async_copy(
            bmap_h.at[pl.ds(0, SBK)],
            bmbuf.at[pl.ds(par * SBK, SBK)], sem.at[3 + par]).wait()
        if sw + 1 < NW:
            issue_sb(sw + 1, (sw + 1) & 1)
        csw = jnp.max(cnt_v[pl.ds(sw * 16, 16)])
        scan_sb(par * SBK, sw, 0, csw)

        def sbf(sb, carry):
            pltpu.async_copy(
                bmap_h.at[pl.ds(sw * SLOTS + (sb + 1) * SBK, SBK)],
                bmbuf.at[pl.ds(par * SBK, SBK)], sem.at[3 + par]).wait()
            scan_sb(par * SBK, sw, (sb + 1) * SBK, csw)
            return carry

        lax.fori_loop(0, (csw + SBK - 1) // SBK - 1, sbf, 0)

    for ch in range(BPW // 128):
        cu = pltpu.async_copy(
            dense_h.at[pos_all.at[pl.ds(ch * 128, 128)]], gu, sem.at[0])
        ci = pltpu.async_copy(
            dense_h.at[pos_all.at[pl.ds(BPW + ch * 128, 128)]], gi,
            sem.at[1])
        cj = pltpu.async_copy(
            dense_h.at[pos_all.at[pl.ds(2 * BPW + ch * 128, 128)]], gj,
            sem.at[2])
        cu.wait()
        ci.wait()
        cj.wait()

        def group(g, carry):
            rows = g * 16 + _IOTA()
            acc_i = jnp.zeros((16,), jnp.float32)
            acc_j = jnp.zeros((16,), jnp.float32)
            for c in range(D):
                col = jnp.full((16,), c, jnp.int32)
                uc = plsc.load_gather(gu, [rows, col])
                vic = plsc.load_gather(gi, [rows, col])
                vjc = plsc.load_gather(gj, [rows, col])
                acc_i = acc_i + uc * vic
                acc_j = acc_j + uc * vjc
            po_i[pl.ds(ch * 128 + g * 16, 16)] = acc_i
            po_j[pl.ds(ch * 128 + g * 16, 16)] = acc_j
            return carry

        lax.fori_loop(0, 8, group, 0)

    pltpu.sync_copy(po_i, pi_h.at[pl.ds(base, BPW)])
    pltpu.sync_copy(po_j, pj_h.at[pl.ds(base, BPW)])


def kernel(user, item_i, item_j, embed_user, embed_item):
    mesh = plsc.VectorSubcoreMesh(core_axis_name="c", subcore_axis_name="s")
    eu_t = embed_user.T
    ei_t = embed_item.T

    def listset():
        return [
            pltpu.VMEM((C + 32,), jnp.int32),
            pltpu.VMEM((C + 32,), jnp.int32),
            pltpu.VMEM((SB * SBC,), jnp.int32),
            pltpu.VMEM((SB * SBC,), jnp.int32),
            pltpu.VMEM((SBC + 16,), jnp.int32),
            pltpu.VMEM((SBC + 16,), jnp.int32),
        ]

    stage1 = pl.kernel(
        _stage1_body,
        mesh=mesh,
        compiler_params=pltpu.CompilerParams(
            needs_layout_passes=False, use_tc_tiling_on_sc=True),
        out_type=(
            jax.ShapeDtypeStruct((NW * SLOTS, 128), jnp.float32),
            jax.ShapeDtypeStruct((NW * SLOTS,), jnp.int32),
            jax.ShapeDtypeStruct((NW * 16,), jnp.int32),
        ),
        scratch_types=[
            pltpu.VMEM((1024,), jnp.int32),
            pltpu.VMEM((NR, D, SLW), jnp.float32),
            pltpu.VMEM((D, 64), jnp.float32),
            pltpu.VMEM((2, 128, 128), jnp.float32),
            pltpu.VMEM((256,), jnp.int32),
        ] + listset() + listset() + [
            pltpu.SemaphoreType.DMA,
            pltpu.SemaphoreType.DMA((NR,)),
            pltpu.SemaphoreType.DMA((2,)),
        ],
    )
    dense, bmap, counts = stage1(user, item_i, item_j, eu_t, ei_t)

    stage2 = pl.kernel(
        _stage2_body,
        mesh=mesh,
        compiler_params=pltpu.CompilerParams(
            needs_layout_passes=False, use_tc_tiling_on_sc=True),
        out_type=(
            jax.ShapeDtypeStruct((B,), jnp.float32),
            jax.ShapeDtypeStruct((B,), jnp.float32),
        ),
        scratch_types=[
            pltpu.VMEM((NW * 16,), jnp.int32),
            pltpu.VMEM((2 * 4096,), jnp.int32),
            pltpu.VMEM((3 * BPW,), jnp.int32),
            pltpu.VMEM((128, 128), jnp.float32),
            pltpu.VMEM((128, 128), jnp.float32),
            pltpu.VMEM((128, 128), jnp.float32),
            pltpu.VMEM((BPW,), jnp.float32),
            pltpu.VMEM((BPW,), jnp.float32),
            pltpu.SemaphoreType.DMA((5,)),
        ],
    )
    return stage2(dense, bmap, counts)


# speedup vs baseline: 3.9331x; 1.0736x over previous
import jax
import jax.numpy as jnp
from jax import lax
from jax.experimental import pallas as pl
from jax.experimental.pallas import tpu as pltpu
from jax.experimental.pallas import tpu_sc as plsc

B = 16384
D = 64
V = 1000000
NW = 32
BPW = B // NW
BCOLS = 244
NR = 4
SLW = 256
C = 1024
SB = 32
SBC = 64
PC = 32
SLOTS = 81920
DV = 1 << 28

_IOTA = lambda: lax.iota(jnp.int32, 16)


def _bucket(idx_hbm, idxbuf, wl_v, wl_b, done, lo, hi, sem):
    pltpu.async_copy(
        idx_hbm.at[pl.ds(0, 1024)], idxbuf.at[pl.ds(0, 1024)], sem.at[0])

    def block(blk, carry):
        par = lax.rem(blk, 2)
        pltpu.make_async_copy(
            idx_hbm.at[pl.ds(0, 1024)], idxbuf.at[pl.ds(par * 1024, 1024)],
            sem.at[par]).wait()

        @pl.when(blk + 1 < B // 1024)
        def _():
            par2 = lax.rem(blk + 1, 2)
            pltpu.async_copy(
                idx_hbm.at[pl.ds((blk + 1) * 1024, 1024)],
                idxbuf.at[pl.ds(par2 * 1024, 1024)], sem.at[par2])

        def chunk(k, carry):
            off, seen = carry
            iv = idxbuf[pl.ds(par * 1024 + k * 16, 16)]
            bv = blk * 1024 + k * 16 + _IOTA()
            m = (iv >= lo) & (iv < hi)
            mcnt = plsc.all_reduce_population_count(m)[0]
            fast = (done == 0) & (off + 16 <= C)

            def fastf(_):
                @pl.when(mcnt > 0)
                def _():
                    plsc.store_compressed(
                        wl_v.at[pl.ds(off, 16)], iv, mask=m)
                    plsc.store_compressed(
                        wl_b.at[pl.ds(off, 16)], bv, mask=m)

                return mcnt

            def slowf(_):
                rank = seen + plsc.cumsum(jnp.where(m, 1, 0))
                keep = m & (rank > done) & (rank <= done + C)
                kcnt = plsc.all_reduce_population_count(keep)[0]

                @pl.when(kcnt > 0)
                def _():
                    plsc.store_compressed(
                        wl_v.at[pl.ds(off, 16)], iv, mask=keep)
                    plsc.store_compressed(
                        wl_b.at[pl.ds(off, 16)], bv, mask=keep)

                return kcnt

            adv = lax.cond(fast, fastf, slowf, 0)
            return off + adv, seen + mcnt

        return lax.fori_loop(0, 64, chunk, carry)

    return lax.fori_loop(0, B // 1024, block, (0, 0))


def _split(wl_v, n, lo, sb_l, sb_m):
    sent = jnp.full((16,), jnp.int32(DV), jnp.int32)

    def initc(t, carry):
        sb_l[pl.ds(t * 16, 16)] = sent
        return carry

    lax.fori_loop(0, SB * SBC // 16, initc, 0)

    def chunk(q, offs):
        lanes = wl_v[pl.ds(q * 16, 16)] - lo
        live = (q * 16 + _IOTA()) < n
        sub = jnp.clip(lanes >> 10, 0, SB - 1)
        mvec = q * 16 + _IOTA()
        out = []
        for s in range(SB):
            off_s = offs[s]
            msk = live & (sub == s)
            cnt = plsc.all_reduce_population_count(msk)[0]

            @pl.when((cnt > 0) & (off_s <= SBC - 16))
            def _(s=s, off_s=off_s, msk=msk):
                plsc.store_compressed(
                    sb_l.at[pl.ds(s * SBC + off_s, 16)], lanes, mask=msk)
                plsc.store_compressed(
                    sb_m.at[pl.ds(s * SBC + off_s, 16)], mvec, mask=msk)

            out.append(off_s + cnt)
        return tuple(out)

    offs = lax.fori_loop(0, (n + 15) // 16, chunk, (0,) * SB)
    ovf = offs[0] > (SBC - 16)
    for s in range(1, SB):
        ovf = ovf | (offs[s] > (SBC - 16))
    return ovf


def _stage1_body(user_h, itemi_h, itemj_h, eu_h, ei_h,
                 dense_out, bmap_out, counts_out,
                 idxbuf, slabs, tailbuf, stage, stage_b,
                 wl_vA, wl_bA, sb_lA, sb_mA, pend_lA, pend_mA,
                 wl_vB, wl_bB, sb_lB, sb_mB, pend_lB, pend_mB,
                 sem_i, sem_s, sem_f):
    w = lax.axis_index("s") * 2 + lax.axis_index("c")
    c0 = w * BCOLS
    lo = c0 * 128
    hi = jnp.where(w == 31, V, (c0 + BCOLS) * 128)
    ncols = BCOLS + jnp.where(w == 31, 4, 0)
    nslab = ncols * 128 // SLW
    dvv = jnp.full((16,), jnp.int32(DV), jnp.int32)

    setA = (wl_vA, wl_bA, sb_lA, sb_mA, pend_lA, pend_mA)
    setB = (wl_vB, wl_bB, sb_lB, sb_mB, pend_lB, pend_mB)

    def init_stage_b(par):
        def ic(t, carry):
            stage_b[pl.ds(par * 128 + t * 16, 16)] = dvv
            return carry

        lax.fori_loop(0, 8, ic, 0)

    def flush(nf):
        par = lax.rem(nf, 2)
        rowoff = w * SLOTS + nf * 128
        c1 = pltpu.async_copy(
            stage.at[par], dense_out.at[pl.ds(rowoff, 128)], sem_f.at[par])
        c2 = pltpu.async_copy(
            stage_b.at[pl.ds(par * 128, 128)],
            bmap_out.at[pl.ds(rowoff, 128)], sem_f.at[par])
        c1.wait()
        c2.wait()
        init_stage_b(lax.rem(nf + 1, 2))

    def append16(tail_src, rvec, lin, msk, livecnt, bvals, soff, nf):
        need = (soff + 16) > 128

        @pl.when(need)
        def _():
            flush(nf)

        soff2 = jnp.where(need, 0, soff)
        nf2 = nf + jnp.where(need, 1, 0)
        par = lax.rem(nf2, 2)
        pos = plsc.cumsum(jnp.where(msk, 1, 0)) - 1
        rowvec = jnp.clip(soff2 + pos, 0, 127)
        for c in range(D):
            col = jnp.full((16,), c, jnp.int32)
            if tail_src is None:
                vals = plsc.load_gather(slabs, [rvec, col, lin], mask=msk)
            else:
                vals = plsc.load_gather(tail_src, [col, lin], mask=msk)
            plsc.store_scatter(stage.at[par], [rowvec, col], vals, mask=msk)
        plsc.store_scatter(stage_b, [par * 128 + rowvec], bvals, mask=msk)
        return soff2 + livecnt, nf2

    def run_pass(tab_hbm, lists, fstate):
        nl = len(lists)

        def issue_slab(s):
            r = lax.rem(s, NR)
            lane0 = pl.multiple_of(c0 * 128 + s * SLW, 128)
            pltpu.async_copy(
                tab_hbm.at[:, pl.ds(lane0, SLW)],
                slabs.at[r], sem_s.at[r])

        def round_body(carry):
            fc = (carry[2 * nl], carry[2 * nl + 1])
            states = []
            for li, (idx_hbm, roff, sset) in enumerate(lists):
                done = carry[2 * li]
                (wl_v, wl_b, sb_l, sb_m, pend_l, pend_m) = sset
                n, total = _bucket(idx_hbm, idxbuf, wl_v, wl_b, done, lo, hi,
                                   sem_i)
                ovf = _split(wl_v, n, lo, sb_l, sb_m)
                states.append((n, total, ovf))

            for kk in range(NR - 1):
                @pl.when(kk < nslab)
                def _(kk=kk):
                    issue_slab(jnp.int32(kk))

            def sbody(s, fc2):
                soff, nf = fc2

                @pl.when(s + NR - 1 < nslab)
                def _():
                    issue_slab(s + NR - 1)

                r = lax.rem(s, NR)
                pltpu.make_async_copy(
                    tab_hbm.at[:, pl.ds(0, SLW)],
                    slabs.at[r], sem_s.at[r]).wait()
                slab_l0 = s * SLW
                sw = s >> 2
                rvec = jnp.full((16,), 0, jnp.int32) + r

                for li, (idx_hbm, roff, sset) in enumerate(lists):
                    (wl_v, wl_b, sb_l, sb_m, pend_l, pend_m) = sset
                    n, total, ovf = states[li]

                    def pchunk(q, poff):
                        lanes = sb_l[pl.ds(sw * SBC + q * 16, 16)]
                        mv = sb_m[pl.ds(sw * SBC + q * 16, 16)]
                        rel = lanes - slab_l0
                        msk = (rel >= 0) & (rel < SLW)
                        cnt = plsc.all_reduce_population_count(msk)[0]

                        @pl.when((cnt > 0) & (poff <= PC - 16))
                        def _():
                            plsc.store_compressed(
                                pend_l.at[pl.ds(poff, 16)], rel, mask=msk)
                            plsc.store_compressed(
                                pend_m.at[pl.ds(poff, 16)], mv, mask=msk)

                        return poff + cnt

                    poff = lax.fori_loop(0, SBC // 16, pchunk, 0)
                    use_slow = ovf | (poff > PC - 16)
                    neff = jnp.where(use_slow, 0, poff)

                    def fblock(pb, fc3):
                        soff3, nf3 = fc3
                        left = neff - pb * 16
                        livemask = _IOTA() < left
                        livecnt = jnp.minimum(left, 16)
                        rel = pend_l[pl.ds(pb * 16, 16)]
                        mvq = pend_m[pl.ds(pb * 16, 16)]
                        bq = plsc.load_gather(
                            wl_b, [jnp.clip(mvq, 0, C + 15)]) + roff
                        return append16(None, rvec, rel & (SLW - 1),
                                        livemask, livecnt, bq, soff3, nf3)

                    soff, nf = lax.fori_loop(0, (neff + 15) // 16, fblock,
                                             (soff, nf))

                    def sc(q, fc3):
                        soff3, nf3 = fc3
                        iv = wl_v[pl.ds(q * 16, 16)]
                        live = (q * 16 + _IOTA()) < n
                        rel = iv - lo - slab_l0
                        msk = live & (rel >= 0) & (rel < SLW)
                        cnt = plsc.all_reduce_population_count(msk)[0]

                        def go(fc4):
                            soff4, nf4 = fc4
                            bq = wl_b[pl.ds(q * 16, 16)] + roff
                            return append16(None, rvec, rel & (SLW - 1), msk,
                                            cnt, bq, soff4, nf4)

                        return lax.cond(cnt > 0, go, lambda fc4: fc4,
                                        (soff3, nf3))

                    nsc = jnp.where(use_slow, (n + 15) // 16, 0)
                    soff, nf = lax.fori_loop(0, nsc, sc, (soff, nf))

                return soff, nf

            fc = lax.fori_loop(0, nslab, sbody, fc)

            def tailf(fc2):
                pltpu.async_copy(
                    tab_hbm.at[:, pl.ds((V // 128) * 128, 64)],
                    tailbuf, sem_s.at[0]).wait()
                tail_l0 = (V // 128) * 128 - lo
                for li, (idx_hbm, roff, sset) in enumerate(lists):
                    (wl_v, wl_b, sb_l, sb_m, pend_l, pend_m) = sset
                    n, total, ovf = states[li]

                    def tc(q, fc3):
                        soff3, nf3 = fc3
                        iv = wl_v[pl.ds(q * 16, 16)]
                        live = (q * 16 + _IOTA()) < n
                        rel = iv - lo - tail_l0
                        msk = live & (rel >= 0) & (rel < 64)
                        cnt = plsc.all_reduce_population_count(msk)[0]

                        def go(fc4):
                            soff4, nf4 = fc4
                            bq = wl_b[pl.ds(q * 16, 16)] + roff
                            return append16(tailbuf, None, rel & 63, msk,
                                            cnt, bq, soff4, nf4)

                        return lax.cond(cnt > 0, go, lambda fc4: fc4,
                                        (soff3, nf3))

                    fc2 = lax.fori_loop(0, (n + 15) // 16, tc, fc2)
                return fc2

            fc = lax.cond(w == 31, tailf, lambda fc2: fc2, fc)

            out_carry = []
            for li in range(nl):
                n, total, ovf = states[li]
                out_carry.extend([carry[2 * li] + n, total])
            return tuple(out_carry) + fc

        def cond(carry):
            more = carry[0] < carry[1]
            for li in range(1, nl):
                more = more | (carry[2 * li] < carry[2 * li + 1])
            return more

        fin = lax.while_loop(cond, round_body,
                             tuple([0, 1] * nl) + tuple(fstate))
        return fin[2 * nl], fin[2 * nl + 1]

    init_stage_b(0)
    fstate = (0, 0)
    fstate = run_pass(eu_h, ((user_h, 0, setA),), fstate)
    fstate = run_pass(ei_h, ((itemi_h, B, setA),
                             (itemj_h, 2 * B, setB)), fstate)
    soff, nf = fstate

    @pl.when(soff > 0)
    def _():
        flush(nf)

    nf_tot = nf + jnp.where(soff > 0, 1, 0)

    cv = jnp.where(_IOTA() == 0, nf_tot * 128, 0)
    pend_lA[pl.ds(0, 16)] = cv
    pltpu.sync_copy(pend_lA.at[pl.ds(0, 16)],
                    counts_out.at[pl.ds(w * 16, 16)])


def _stage2_body(dense_h, bmap_h, counts_h, pi_h, pj_h,
                 cnt_v, bmbuf, pos_all,
                 gu, gi, gj, po_i, po_j, sem):
    w = lax.axis_index("s") * 2 + lax.axis_index("c")
    base = w * BPW
    SBK = 4096

    pltpu.sync_copy(counts_h, cnt_v)

    zv = jnp.zeros((16,), jnp.int32)

    def pinit(t, carry):
        pos_all[pl.ds(t * 16, 16)] = zv
        return carry

    lax.fori_loop(0, 3 * BPW // 16, pinit, 0)

    def scan_sb(pbase, sw, sb0, csw):
        def chunk(k, carry):
            bv = bmbuf[pl.ds(pbase + k * 16, 16)]
            live = (sb0 + k * 16 + _IOTA()) < csw
            live = live & (bv < jnp.int32(3 * B))
            reg = jnp.clip(bv >> 14, 0, 2)
            b = bv & (B - 1)
            mine = live & (b >= base) & (b < base + BPW)
            slotv = sw * SLOTS + sb0 + k * 16 + _IOTA()
            rowl = reg * BPW + (b - base)
            plsc.store_scatter(pos_all, [rowl], slotv, mask=mine)
            return carry

        nch = (jnp.clip(csw - sb0, 0, SBK) + 15) // 16
        lax.fori_loop(0, nch, chunk, 0)

    def issue_sb(sw, par):
        pltpu.async_copy(
            bmap_h.at[pl.ds(sw * SLOTS, SBK)],
            bmbuf.at[pl.ds(par * SBK, SBK)], sem.at[6 + par])

    issue_sb(0, 0)
    for sw in range(NW):
        par = sw & 1
        pltpu.make_async_copy(
            bmap_h.at[pl.ds(0, SBK)],
            bmbuf.at[pl.ds(par * SBK, SBK)], sem.at[6 + par]).wait()
        if sw + 1 < NW:
            issue_sb(sw + 1, (sw + 1) & 1)
        csw = jnp.max(cnt_v[pl.ds(sw * 16, 16)])
        scan_sb(par * SBK, sw, 0, csw)

        def sbf(sb, carry):
            pltpu.async_copy(
                bmap_h.at[pl.ds(sw * SLOTS + (sb + 1) * SBK, SBK)],
                bmbuf.at[pl.ds(par * SBK, SBK)], sem.at[6 + par]).wait()
            scan_sb(par * SBK, sw, (sb + 1) * SBK, csw)
            return carry

        lax.fori_loop(0, (csw + SBK - 1) // SBK - 1, sbf, 0)

    def issue_ch(ch):
        par = ch & 1
        hs = []
        for k, (poff, buf) in enumerate(((0, gu), (BPW, gi), (2 * BPW, gj))):
            hs.append(pltpu.async_copy(
                dense_h.at[pos_all.at[pl.ds(poff + ch * 128, 128)]],
                buf.at[par], sem.at[3 * par + k]))
        return hs

    handles = {0: issue_ch(0)}
    for ch in range(BPW // 128):
        if ch + 1 < BPW // 128:
            handles[ch + 1] = issue_ch(ch + 1)
        for h in handles.pop(ch):
            h.wait()
        par = ch & 1

        def group(g, carry):
            rows = g * 16 + _IOTA()
            acc_i = jnp.zeros((16,), jnp.float32)
            acc_j = jnp.zeros((16,), jnp.float32)
            for c in range(D):
                col = jnp.full((16,), c, jnp.int32)
                uc = plsc.load_gather(gu.at[par], [rows, col])
                vic = plsc.load_gather(gi.at[par], [rows, col])
                vjc = plsc.load_gather(gj.at[par], [rows, col])
                acc_i = acc_i + uc * vic
                acc_j = acc_j + uc * vjc
            po_i[pl.ds(ch * 128 + g * 16, 16)] = acc_i
            po_j[pl.ds(ch * 128 + g * 16, 16)] = acc_j
            return carry

        lax.fori_loop(0, 8, group, 0)

    pltpu.sync_copy(po_i, pi_h.at[pl.ds(base, BPW)])
    pltpu.sync_copy(po_j, pj_h.at[pl.ds(base, BPW)])


def kernel(user, item_i, item_j, embed_user, embed_item):
    mesh = plsc.VectorSubcoreMesh(core_axis_name="c", subcore_axis_name="s")
    eu_t = embed_user.T
    ei_t = embed_item.T

    def listset():
        return [
            pltpu.VMEM((C + 32,), jnp.int32),
            pltpu.VMEM((C + 32,), jnp.int32),
            pltpu.VMEM((SB * SBC,), jnp.int32),
            pltpu.VMEM((SB * SBC,), jnp.int32),
            pltpu.VMEM((SBC + 16,), jnp.int32),
            pltpu.VMEM((SBC + 16,), jnp.int32),
        ]

    stage1 = pl.kernel(
        _stage1_body,
        mesh=mesh,
        compiler_params=pltpu.CompilerParams(
            needs_layout_passes=False, use_tc_tiling_on_sc=True),
        out_type=(
            jax.ShapeDtypeStruct((NW * SLOTS, 128), jnp.float32),
            jax.ShapeDtypeStruct((NW * SLOTS,), jnp.int32),
            jax.ShapeDtypeStruct((NW * 16,), jnp.int32),
        ),
        scratch_types=[
            pltpu.VMEM((2048,), jnp.int32),
            pltpu.VMEM((NR, D, SLW), jnp.float32),
            pltpu.VMEM((D, 64), jnp.float32),
            pltpu.VMEM((2, 128, 128), jnp.float32),
            pltpu.VMEM((256,), jnp.int32),
        ] + listset() + listset() + [
            pltpu.SemaphoreType.DMA((2,)),
            pltpu.SemaphoreType.DMA((NR,)),
            pltpu.SemaphoreType.DMA((2,)),
        ],
    )
    dense, bmap, counts = stage1(user, item_i, item_j, eu_t, ei_t)

    stage2 = pl.kernel(
        _stage2_body,
        mesh=mesh,
        compiler_params=pltpu.CompilerParams(
            needs_layout_passes=False, use_tc_tiling_on_sc=True),
        out_type=(
            jax.ShapeDtypeStruct((B,), jnp.float32),
            jax.ShapeDtypeStruct((B,), jnp.float32),
        ),
        scratch_types=[
            pltpu.VMEM((NW * 16,), jnp.int32),
            pltpu.VMEM((2 * 4096,), jnp.int32),
            pltpu.VMEM((3 * BPW,), jnp.int32),
            pltpu.VMEM((2, 128, 128), jnp.float32),
            pltpu.VMEM((2, 128, 128), jnp.float32),
            pltpu.VMEM((2, 128, 128), jnp.float32),
            pltpu.VMEM((BPW,), jnp.float32),
            pltpu.VMEM((BPW,), jnp.float32),
            pltpu.SemaphoreType.DMA((8,)),
        ],
    )
    return stage2(dense, bmap, counts)
